# baseline (speedup 1.0000x reference)
import functools

import jax
import jax.numpy as jnp
from jax import lax
from jax.experimental import pallas as pl
from jax.experimental.pallas import tpu as pltpu
from jax.experimental.pallas import tpu_sc as plsc

N = 50000
E = 800000
P = 51200
D = 16
NC, NS = 2, 16
NW = NC * NS
E2 = 802816
ER = E2 // 128
DROW = 51168
RPW = ER // NW
RC = 7
C = RC * 128
G = RPW // RC
SL = P // NS
ZR = 200
R = 2000


def _mesh():
    return plsc.VectorSubcoreMesh(
        core_axis_name="c", subcore_axis_name="s", num_cores=NC, num_subcores=NS
    )


_SC_PARAMS = pltpu.CompilerParams(use_tc_tiling_on_sc=False)


def _deg_body(dst_hbm, out_hbm,
              ib0, ib1, ib2, ib3, ones_v, zb,
              si0, si1, si2, si3, ss0, ss1, accum_sh):
    c = lax.axis_index("c")
    s = lax.axis_index("s")
    ib = [ib0, ib1, ib2, ib3]
    si = [si0, si1, si2, si3]
    ss = [ss0, ss1]

    def _load(j, slot):
        pltpu.async_copy(dst_hbm.at[pl.ds(base + j * RC, RC)], ib[slot],
                         si[slot])

    def _wait_load(j, slot):
        pltpu.make_async_copy(
            dst_hbm.at[pl.ds(base + j * RC, RC)], ib[slot], si[slot]).wait()

    def _scat(slot, sem):
        for r in range(RC):
            pltpu.async_copy(ones_v.at[r], accum_sh.at[ib[slot].at[r]],
                             ss[sem], add=True)

    def _wait_scat(slot):
        for r in range(RC):
            pltpu.make_async_copy(
                ones_v.at[r], accum_sh.at[ib[0].at[r]], ss[slot]).wait()

    base = (c * NS + s) * RPW
    for b in (0, 1):
        _load(b, b)

    for i in range(RC):
        for k in range(8):
            ones_v[i, pl.ds(k * 16, 16)] = jnp.full((16,), 1.0, jnp.float32)

    def zfill(i, carry):
        zb[pl.ds(i * 16, 16)] = jnp.zeros((16,), jnp.float32)
        return carry

    lax.fori_loop(0, SL // 16, zfill, 0)
    pltpu.sync_copy(zb, accum_sh.at[pl.ds(s * SL, SL)])
    plsc.subcore_barrier()

    def quad(g, carry):
        for b in (0, 1, 2, 3):
            j = 4 * g + b
            if b < 2:

                @pl.when(g >= 1)
                def _():
                    _wait_scat(b % 2)
            else:
                _wait_scat(b % 2)
            @pl.when(j + 2 < G)
            def _():
                _load(j + 2, (b + 2) % 4)

            _wait_load(j, b)
            _scat(b, b % 2)
        return carry

    lax.fori_loop(0, G // 4, quad, 0)
    if G % 4:
        _wait_scat(0)
        _wait_load(G - 1, 0)
        _scat(0, 0)
    _wait_scat(0)
    _wait_scat(1)

    plsc.subcore_barrier()
    pltpu.sync_copy(accum_sh.at[pl.ds(s * SL, SL)],
                    out_hbm.at[pl.ds(c * P + s * SL, SL)])


def _agg_body(table_hbm, src_hbm, dst_hbm, out_hbm,
              sbuf0, sbuf1, dbuf0, dbuf1, rows0, rows1, zb,
              ssrc0, ssrc1, sd0, sd1, sg0, sg1, ss0, ss1, accum_sh):
    c = lax.axis_index("c")
    s = lax.axis_index("s")
    sbuf = [sbuf0, sbuf1]
    dbuf = [dbuf0, dbuf1]
    rows = [rows0, rows1]
    ssrc = [ssrc0, ssrc1]
    sd = [sd0, sd1]
    sg = [sg0, sg1]
    ss = [ss0, ss1]
    base = (c * NS + s) * RPW

    for b in (0, 1):
        pltpu.async_copy(src_hbm.at[pl.ds(base + b * RC, RC)], sbuf[b],
                         ssrc[b])

    def zfill(i, carry):
        zb[i, :] = jnp.zeros((D,), jnp.float32)
        return carry

    lax.fori_loop(0, ZR, zfill, 0)

    def zcopy(k, carry):
        pltpu.sync_copy(zb, accum_sh.at[pl.ds(s * SL + k * ZR, ZR)])
        return carry

    lax.fori_loop(0, SL // ZR, zcopy, 0)
    plsc.subcore_barrier()

    def _wait_scatter(b):
        for r in range(RC):
            pltpu.make_async_copy(
                rows[b].at[r], accum_sh.at[dbuf[b].at[r]], ss[b]).wait()

    def _wait_src(b, j):
        pltpu.make_async_copy(
            src_hbm.at[pl.ds(base + j * RC, RC)], sbuf[b], ssrc[b]).wait()

    def pair(g, carry):
        for b in (0, 1):
            j = 2 * g + b

            @pl.when(g >= 1)
            def _():
                _wait_scatter(b)

            _wait_src(b, j)
            for r in range(RC):
                pltpu.async_copy(table_hbm.at[sbuf[b].at[r]], rows[b].at[r],
                                 sg[b])
            dld = pltpu.async_copy(
                dst_hbm.at[pl.ds(base + j * RC, RC)], dbuf[b], sd[b])
            for r in range(RC):
                pltpu.make_async_copy(table_hbm.at[sbuf[b].at[r]],
                                      rows[b].at[r], sg[b]).wait()
            dld.wait()
            for r in range(RC):
                pltpu.async_copy(rows[b].at[r], accum_sh.at[dbuf[b].at[r]],
                                 ss[b], add=True)

            @pl.when(j + 2 < G)
            def _():
                pltpu.async_copy(
                    src_hbm.at[pl.ds(base + (j + 2) * RC, RC)], sbuf[b],
                    ssrc[b])
        return carry

    lax.fori_loop(0, G // 2, pair, 0)

    if G % 2:
        _wait_scatter(0)
        _wait_src(0, G - 1)
        for r in range(RC):
            pltpu.async_copy(table_hbm.at[sbuf[0].at[r]], rows[0].at[r],
                             sg[0])
        dld = pltpu.async_copy(
            dst_hbm.at[pl.ds(base + (G - 1) * RC, RC)], dbuf[0], sd[0])
        for r in range(RC):
            pltpu.make_async_copy(table_hbm.at[sbuf[0].at[r]], rows[0].at[r],
                                  sg[0]).wait()
        dld.wait()
        for r in range(RC):
            pltpu.async_copy(rows[0].at[r], accum_sh.at[dbuf[0].at[r]],
                             ss[0], add=True)
    _wait_scatter(0)
    _wait_scatter(1)

    plsc.subcore_barrier()
    pltpu.sync_copy(accum_sh.at[pl.ds(s * SL, SL)],
                    out_hbm.at[pl.ds(c * P + s * SL, SL)])


def _deg(dst):
    k = functools.partial(
        pl.kernel,
        out_type=jax.ShapeDtypeStruct((NC * P,), jnp.float32),
        mesh=_mesh(),
        compiler_params=_SC_PARAMS,
        scratch_types=[
            pltpu.VMEM((RC, 128), jnp.int32),
            pltpu.VMEM((RC, 128), jnp.int32),
            pltpu.VMEM((RC, 128), jnp.int32),
            pltpu.VMEM((RC, 128), jnp.int32),
            pltpu.VMEM((RC, 128), jnp.float32),
            pltpu.VMEM((SL,), jnp.float32),
            pltpu.SemaphoreType.DMA,
            pltpu.SemaphoreType.DMA,
            pltpu.SemaphoreType.DMA,
            pltpu.SemaphoreType.DMA,
            pltpu.SemaphoreType.DMA,
            pltpu.SemaphoreType.DMA,
            pltpu.VMEM_SHARED((P,), jnp.float32),
        ],
    )(_deg_body)
    return k(dst)


def _agg(table, src, dst):
    k = functools.partial(
        pl.kernel,
        out_type=jax.ShapeDtypeStruct((NC * P, D), jnp.float32),
        mesh=_mesh(),
        compiler_params=_SC_PARAMS,
        scratch_types=[
            pltpu.VMEM((RC, 128), jnp.int32),
            pltpu.VMEM((RC, 128), jnp.int32),
            pltpu.VMEM((RC, 128), jnp.int32),
            pltpu.VMEM((RC, 128), jnp.int32),
            pltpu.VMEM((RC, 128, D), jnp.float32),
            pltpu.VMEM((RC, 128, D), jnp.float32),
            pltpu.VMEM((ZR, D), jnp.float32),
            pltpu.SemaphoreType.DMA,
            pltpu.SemaphoreType.DMA,
            pltpu.SemaphoreType.DMA,
            pltpu.SemaphoreType.DMA,
            pltpu.SemaphoreType.DMA,
            pltpu.SemaphoreType.DMA,
            pltpu.SemaphoreType.DMA,
            pltpu.SemaphoreType.DMA,
            pltpu.VMEM_SHARED((P, D), jnp.float32),
        ],
    )(_agg_body)
    return k(table, src, dst)


WR = N * D // 128
PR = P * D // 128
DO = 20


def _kd_body(dp_ref, dinv_ref):
    dinv_ref[...] = lax.rsqrt(dp_ref[0] + dp_ref[1] + 1.0)


def _kd(degp):
    return pl.pallas_call(
        _kd_body,
        grid=(1,),
        in_specs=[pl.BlockSpec((2, P // 128, 128), lambda i: (0, 0, 0))],
        out_specs=pl.BlockSpec((P // 128, 128), lambda i: (0, 0)),
        out_shape=jax.ShapeDtypeStruct((P // 128, 128), jnp.float32),
    )(degp)


def _full(shape):
    return pl.BlockSpec(shape, lambda: tuple(0 for _ in shape))


def _tc1_body(x8_ref, w_ref, dinv_ref, hs_ref):
    h = jnp.dot(x8_ref[...], w_ref[...], preferred_element_type=jnp.float32)
    hs_ref[...] = dinv_ref[...] * h


def _tc1(x8, W1big, dinv16w):
    return pl.pallas_call(
        _tc1_body,
        in_specs=[_full(x8.shape), _full(W1big.shape), _full((WR, 128))],
        out_specs=_full((WR, 128)),
        out_shape=jax.ShapeDtypeStruct((WR, 128), jnp.float32),
    )(x8, W1big, dinv16w)


def _tc2_body(aw_ref, hs_ref, dinv_ref, b_ref, out_ref):
    dinv = dinv_ref[...]
    agg = aw_ref[0, :WR] + aw_ref[1, :WR] + hs_ref[...]
    pre = dinv * agg + b_ref[...]
    out_ref[...] = dinv * jnp.maximum(pre, 0.0)


def _tc2(a1w, hs1w, dinv16w, b1w):
    return pl.pallas_call(
        _tc2_body,
        in_specs=[_full(a1w.shape), _full((WR, 128)), _full((WR, 128)),
                  _full((1, 128))],
        out_specs=_full((WR, 128)),
        out_shape=jax.ShapeDtypeStruct((WR, 128), jnp.float32),
    )(a1w, hs1w, dinv16w, b1w)


def _tc3_body(qw_ref, hs_ref, dinv_ref, w_ref, b_ref, gs_ref, out_ref):
    agg = qw_ref[0, :WR] + qw_ref[1, :WR] + hs_ref[...]
    aggv = dinv_ref[...] * agg
    o = jnp.dot(aggv, w_ref[...], preferred_element_type=jnp.float32)
    o = o + b_ref[...]
    m = jnp.max(o, axis=1, keepdims=True)
    e = jnp.exp(o - m)
    s = jnp.dot(e, gs_ref[...], preferred_element_type=jnp.float32)
    out_ref[...] = (o - m) - jnp.log(s)


def _tc3(a2w, hs2w, dinv16w, W2big, b2w, gs):
    return pl.pallas_call(
        _tc3_body,
        in_specs=[_full(a2w.shape), _full((WR, 128)), _full((WR, 128)),
                  _full(W2big.shape), _full((1, 8 * DO)), _full(gs.shape)],
        out_specs=_full((WR, 8 * DO)),
        out_shape=jax.ShapeDtypeStruct((WR, 8 * DO), jnp.float32),
    )(a2w, hs2w, dinv16w, W2big, b2w, gs)


def kernel(x, edge_index, W1, b1, W2, b2):
    ei = edge_index.astype(jnp.int32)
    pad_s = jnp.zeros((E2 - E,), jnp.int32)
    pad_d = jnp.full((E2 - E,), DROW, jnp.int32)
    src = jnp.concatenate([ei[0], pad_s]).reshape(ER, 128)
    dst = jnp.concatenate([ei[1], pad_d]).reshape(ER, 128)

    x8 = x.reshape(WR, 800)
    degp = _deg(dst)
    dinv_flat = _kd(degp.reshape(2, P // 128, 128))
    dinv16w = jnp.repeat(dinv_flat.reshape(P)[:N], D).reshape(WR, 128)

    eye8 = jnp.eye(8, dtype=jnp.float32)
    W1big = jnp.einsum("ab,ij->aibj", eye8, W1).reshape(8 * 100, 128)
    W2big = jnp.einsum("ab,ij->aibj", eye8, W2).reshape(128, 8 * DO)
    lane = jnp.arange(8 * DO)
    gs = (lane[:, None] // DO == lane[None, :] // DO).astype(jnp.float32)

    hs1w = _tc1(x8, W1big, dinv16w)
    a1 = _agg(hs1w.reshape(N, D), src, dst)
    hs2w = _tc2(a1.reshape(NC, PR, 128), hs1w, dinv16w,
                jnp.tile(b1, 8).reshape(1, 128))
    a2 = _agg(hs2w.reshape(N, D), src, dst)
    ow = _tc3(a2.reshape(NC, PR, 128), hs2w, dinv16w, W2big,
              jnp.tile(b2, 8).reshape(1, 8 * DO), gs)
    return ow.reshape(N, DO)


# speedup vs baseline: 1.0885x; 1.0885x over previous
import functools

import jax
import jax.numpy as jnp
from jax import lax
from jax.experimental import pallas as pl
from jax.experimental.pallas import tpu as pltpu
from jax.experimental.pallas import tpu_sc as plsc

N = 50000
E = 800000
P = 51200
D = 16
NC, NS = 2, 16
NW = NC * NS
EPW = E // NW
C = 1000
G = EPW // C
SL = P // NS
ZR = 200
R = 2000


def _mesh():
    return plsc.VectorSubcoreMesh(
        core_axis_name="c", subcore_axis_name="s", num_cores=NC, num_subcores=NS
    )


_SC_PARAMS = pltpu.CompilerParams(use_tc_tiling_on_sc=False)


def _deg_body(dst_hbm, out_hbm,
              ib0, ib1, ib2, ib3, ones_v, zb,
              si0, si1, si2, si3, ss0, ss1, accum_sh):
    c = lax.axis_index("c")
    s = lax.axis_index("s")
    ib = [ib0, ib1, ib2, ib3]
    si = [si0, si1, si2, si3]
    ss = [ss0, ss1]

    def _load(j, slot):
        pltpu.async_copy(dst_hbm.at[pl.ds(base + j * C, C)], ib[slot], si[slot])

    def _wait_load(j, slot):
        pltpu.make_async_copy(
            dst_hbm.at[pl.ds(base + j * C, C)], ib[slot], si[slot]).wait()

    def _wait_scat(slot):
        pltpu.make_async_copy(
            ones_v.at[pl.ds(0, C)], accum_sh.at[ib[0]], ss[slot]).wait()

    base = (c * NS + s) * EPW
    for b in (0, 1):
        _load(b, b)

    def fill(i, carry):
        ones_v[pl.ds(i * 16, 16)] = jnp.full((16,), 1.0, jnp.float32)
        return carry

    lax.fori_loop(0, ones_v.shape[0] // 16, fill, 0)

    def zfill(i, carry):
        zb[pl.ds(i * 16, 16)] = jnp.zeros((16,), jnp.float32)
        return carry

    lax.fori_loop(0, SL // 16, zfill, 0)
    pltpu.sync_copy(zb, accum_sh.at[pl.ds(s * SL, SL)])
    plsc.subcore_barrier()

    def quad(g, carry):
        for b in (0, 1, 2, 3):
            j = 4 * g + b
            if b < 2:

                @pl.when(g >= 1)
                def _():
                    _wait_scat(b % 2)
            else:
                _wait_scat(b % 2)
            if b == 3:

                @pl.when(g < G // 4 - 1)
                def _():
                    _load(j + 2, (b + 2) % 4)
            else:
                _load(j + 2, (b + 2) % 4)
            _wait_load(j, b)
            pltpu.async_copy(ones_v.at[pl.ds(0, C)], accum_sh.at[ib[b]],
                             ss[b % 2], add=True)
        return carry

    lax.fori_loop(0, G // 4, quad, 0)
    _wait_scat(0)
    _wait_load(G - 1, 0)
    pltpu.async_copy(ones_v.at[pl.ds(0, C)], accum_sh.at[ib[0]], ss[0],
                     add=True)
    _wait_scat(0)
    _wait_scat(1)

    plsc.subcore_barrier()
    pltpu.sync_copy(accum_sh.at[pl.ds(s * SL, SL)],
                    out_hbm.at[pl.ds(c * P + s * SL, SL)])


def _agg_body(table_hbm, src_hbm, dst_hbm, out_hbm,
              sbuf0, sbuf1, dbuf0, dbuf1, rows0, rows1, zb,
              ssrc0, ssrc1, sd0, sd1, sg0, sg1, ss0, ss1, accum_sh):
    c = lax.axis_index("c")
    s = lax.axis_index("s")
    sbuf = [sbuf0, sbuf1]
    dbuf = [dbuf0, dbuf1]
    rows = [rows0, rows1]
    ssrc = [ssrc0, ssrc1]
    sd = [sd0, sd1]
    sg = [sg0, sg1]
    ss = [ss0, ss1]
    base = (c * NS + s) * EPW

    for b in (0, 1):
        pltpu.async_copy(src_hbm.at[pl.ds(base + b * C, C)], sbuf[b], ssrc[b])

    def zfill(i, carry):
        zb[i, :] = jnp.zeros((D,), jnp.float32)
        return carry

    lax.fori_loop(0, ZR, zfill, 0)

    def zcopy(k, carry):
        pltpu.sync_copy(zb, accum_sh.at[pl.ds(s * SL + k * ZR, ZR)])
        return carry

    lax.fori_loop(0, SL // ZR, zcopy, 0)
    plsc.subcore_barrier()

    def _wait_scatter(b):
        pltpu.make_async_copy(rows[b], accum_sh.at[dbuf[b]], ss[b]).wait()

    def _wait_src(b, j):
        pltpu.make_async_copy(
            src_hbm.at[pl.ds(base + j * C, C)], sbuf[b], ssrc[b]).wait()

    def pair(g, carry):
        for b in (0, 1):
            j = 2 * g + b

            @pl.when(g >= 1)
            def _():
                _wait_scatter(b)

            _wait_src(b, j)
            gat = pltpu.async_copy(table_hbm.at[sbuf[b]], rows[b], sg[b])
            dld = pltpu.async_copy(
                dst_hbm.at[pl.ds(base + j * C, C)], dbuf[b], sd[b])
            gat.wait()
            dld.wait()
            pltpu.async_copy(rows[b], accum_sh.at[dbuf[b]], ss[b], add=True)
            if b == 0:
                pltpu.async_copy(
                    src_hbm.at[pl.ds(base + (j + 2) * C, C)], sbuf[b], ssrc[b])
            else:

                @pl.when(g < G // 2 - 1)
                def _():
                    pltpu.async_copy(
                        src_hbm.at[pl.ds(base + (j + 2) * C, C)],
                        sbuf[b], ssrc[b])
        return carry

    lax.fori_loop(0, G // 2, pair, 0)

    _wait_scatter(0)
    _wait_src(0, G - 1)
    gat = pltpu.async_copy(table_hbm.at[sbuf[0]], rows[0], sg[0])
    dld = pltpu.async_copy(
        dst_hbm.at[pl.ds(base + (G - 1) * C, C)], dbuf[0], sd[0])
    gat.wait()
    dld.wait()
    pltpu.async_copy(rows[0], accum_sh.at[dbuf[0]], ss[0], add=True)
    _wait_scatter(0)
    _wait_scatter(1)

    plsc.subcore_barrier()
    pltpu.sync_copy(accum_sh.at[pl.ds(s * SL, SL)],
                    out_hbm.at[pl.ds(c * P + s * SL, SL)])


def _deg(dst):
    k = functools.partial(
        pl.kernel,
        out_type=jax.ShapeDtypeStruct((NC * P,), jnp.float32),
        mesh=_mesh(),
        compiler_params=_SC_PARAMS,
        scratch_types=[
            pltpu.VMEM((C,), jnp.int32),
            pltpu.VMEM((C,), jnp.int32),
            pltpu.VMEM((C,), jnp.int32),
            pltpu.VMEM((C,), jnp.int32),
            pltpu.VMEM((1024,), jnp.float32),
            pltpu.VMEM((SL,), jnp.float32),
            pltpu.SemaphoreType.DMA,
            pltpu.SemaphoreType.DMA,
            pltpu.SemaphoreType.DMA,
            pltpu.SemaphoreType.DMA,
            pltpu.SemaphoreType.DMA,
            pltpu.SemaphoreType.DMA,
            pltpu.VMEM_SHARED((P,), jnp.float32),
        ],
    )(_deg_body)
    return k(dst)


def _agg(table, src, dst):
    k = functools.partial(
        pl.kernel,
        out_type=jax.ShapeDtypeStruct((NC * P, D), jnp.float32),
        mesh=_mesh(),
        compiler_params=_SC_PARAMS,
        scratch_types=[
            pltpu.VMEM((C,), jnp.int32),
            pltpu.VMEM((C,), jnp.int32),
            pltpu.VMEM((C,), jnp.int32),
            pltpu.VMEM((C,), jnp.int32),
            pltpu.VMEM((C, D), jnp.float32),
            pltpu.VMEM((C, D), jnp.float32),
            pltpu.VMEM((ZR, D), jnp.float32),
            pltpu.SemaphoreType.DMA,
            pltpu.SemaphoreType.DMA,
            pltpu.SemaphoreType.DMA,
            pltpu.SemaphoreType.DMA,
            pltpu.SemaphoreType.DMA,
            pltpu.SemaphoreType.DMA,
            pltpu.SemaphoreType.DMA,
            pltpu.SemaphoreType.DMA,
            pltpu.VMEM_SHARED((P, D), jnp.float32),
        ],
    )(_agg_body)
    return k(table, src, dst)


WR = N * D // 128
PR = P * D // 128
DO = 20


def _kd_body(dp_ref, dinv_ref):
    dinv_ref[...] = lax.rsqrt(dp_ref[0] + dp_ref[1] + 1.0)


def _kd(degp):
    return pl.pallas_call(
        _kd_body,
        grid=(1,),
        in_specs=[pl.BlockSpec((2, P // 128, 128), lambda i: (0, 0, 0))],
        out_specs=pl.BlockSpec((P // 128, 128), lambda i: (0, 0)),
        out_shape=jax.ShapeDtypeStruct((P // 128, 128), jnp.float32),
    )(degp)


def _full(shape):
    return pl.BlockSpec(shape, lambda: tuple(0 for _ in shape))


def _tc1_body(x8_ref, w_ref, dinv_ref, hs_ref):
    h = jnp.dot(x8_ref[...], w_ref[...], preferred_element_type=jnp.float32)
    hs_ref[...] = dinv_ref[...] * h


def _tc1(x8, W1big, dinv16w):
    return pl.pallas_call(
        _tc1_body,
        in_specs=[_full(x8.shape), _full(W1big.shape), _full((WR, 128))],
        out_specs=_full((WR, 128)),
        out_shape=jax.ShapeDtypeStruct((WR, 128), jnp.float32),
    )(x8, W1big, dinv16w)


def _tc2_body(aw_ref, hs_ref, dinv_ref, b_ref, out_ref):
    dinv = dinv_ref[...]
    agg = aw_ref[0, :WR] + aw_ref[1, :WR] + hs_ref[...]
    pre = dinv * agg + b_ref[...]
    out_ref[...] = dinv * jnp.maximum(pre, 0.0)


def _tc2(a1w, hs1w, dinv16w, b1w):
    return pl.pallas_call(
        _tc2_body,
        in_specs=[_full(a1w.shape), _full((WR, 128)), _full((WR, 128)),
                  _full((1, 128))],
        out_specs=_full((WR, 128)),
        out_shape=jax.ShapeDtypeStruct((WR, 128), jnp.float32),
    )(a1w, hs1w, dinv16w, b1w)


def _tc3_body(qw_ref, hs_ref, dinv_ref, w_ref, b_ref, gs_ref, out_ref):
    agg = qw_ref[0, :WR] + qw_ref[1, :WR] + hs_ref[...]
    aggv = dinv_ref[...] * agg
    o = jnp.dot(aggv, w_ref[...], preferred_element_type=jnp.float32)
    o = o + b_ref[...]
    m = jnp.max(o, axis=1, keepdims=True)
    e = jnp.exp(o - m)
    s = jnp.dot(e, gs_ref[...], preferred_element_type=jnp.float32)
    out_ref[...] = (o - m) - jnp.log(s)


def _tc3(a2w, hs2w, dinv16w, W2big, b2w, gs):
    return pl.pallas_call(
        _tc3_body,
        in_specs=[_full(a2w.shape), _full((WR, 128)), _full((WR, 128)),
                  _full(W2big.shape), _full((1, 8 * DO)), _full(gs.shape)],
        out_specs=_full((WR, 8 * DO)),
        out_shape=jax.ShapeDtypeStruct((WR, 8 * DO), jnp.float32),
    )(a2w, hs2w, dinv16w, W2big, b2w, gs)


def kernel(x, edge_index, W1, b1, W2, b2):
    ei = edge_index.astype(jnp.int32)
    src, dst = ei[0], ei[1]

    x8 = x.reshape(WR, 800)
    degp = _deg(dst)
    dinv_flat = _kd(degp.reshape(2, P // 128, 128))
    dinv16w = jnp.repeat(dinv_flat.reshape(P)[:N], D).reshape(WR, 128)

    eye8 = jnp.eye(8, dtype=jnp.float32)
    W1big = jnp.einsum("ab,ij->aibj", eye8, W1).reshape(8 * 100, 128)
    W2big = jnp.einsum("ab,ij->aibj", eye8, W2).reshape(128, 8 * DO)
    lane = jnp.arange(8 * DO)
    gs = (lane[:, None] // DO == lane[None, :] // DO).astype(jnp.float32)

    hs1w = _tc1(x8, W1big, dinv16w)
    a1 = _agg(hs1w.reshape(N, D), src, dst)
    hs2w = _tc2(a1.reshape(NC, PR, 128), hs1w, dinv16w,
                jnp.tile(b1, 8).reshape(1, 128))
    a2 = _agg(hs2w.reshape(N, D), src, dst)
    ow = _tc3(a2.reshape(NC, PR, 128), hs2w, dinv16w, W2big,
              jnp.tile(b2, 8).reshape(1, 8 * DO), gs)
    return ow.reshape(N, DO)
